# baseline (speedup 1.0000x reference)
import jax
import jax.numpy as jnp
from jax.experimental import pallas as pl
from jax.experimental.pallas import tpu as pltpu
from jax.experimental.pallas import tpu_sc as plsc

_NC, _NS = 2, 16
_CHUNK = 32
_NBUF = 3
_HEAD = 512
_TAILW = 512


def _sc_gather_ring(table, idx_flat):
    nidx = idx_flat.shape[0]
    width = table.shape[1]
    nw = _NC * _NS
    per_w = nidx // nw
    rounds = per_w // _CHUNK
    mesh = plsc.VectorSubcoreMesh(core_axis_name="c", subcore_axis_name="s")

    scratch = []
    for _ in range(_NBUF):
        scratch += [
            pltpu.VMEM((_CHUNK,), jnp.int32),
            pltpu.VMEM((_CHUNK, width), jnp.float32),
            pltpu.SemaphoreType.DMA,
            pltpu.SemaphoreType.DMA,
        ]

    @pl.kernel(
        out_type=jax.ShapeDtypeStruct((nidx, width), jnp.float32),
        mesh=mesh,
        scratch_types=scratch,
    )
    def gather_kernel(tbl_hbm, idx_hbm, out_hbm, *bufs):
        idx_v = bufs[0::4]
        rows_v = bufs[1::4]
        gsem = bufs[2::4]
        wsem = bufs[3::4]
        wid = jax.lax.axis_index("s") * _NC + jax.lax.axis_index("c")
        base = wid * per_w

        def start_gather(rnd, slot):
            off = base + rnd * _CHUNK
            pltpu.sync_copy(idx_hbm.at[pl.ds(off, _CHUNK)], idx_v[slot])
            pltpu.async_copy(tbl_hbm.at[idx_v[slot]], rows_v[slot], gsem[slot])

        for slot in range(_NBUF):
            start_gather(slot, slot)

        @pl.loop(0, rounds // _NBUF)
        def _(t):
            cur0 = t * _NBUF
            for slot in range(_NBUF):
                cur = cur0 + slot
                off = base + cur * _CHUNK
                pltpu.make_async_copy(
                    tbl_hbm.at[idx_v[slot]], rows_v[slot], gsem[slot]
                ).wait()
                pltpu.async_copy(rows_v[slot], out_hbm.at[pl.ds(off, _CHUNK)], wsem[slot])
                nxt = cur + _NBUF

                @pl.when(nxt < rounds)
                def _():
                    pltpu.make_async_copy(
                        rows_v[slot], out_hbm.at[pl.ds(base, _CHUNK)], wsem[slot]
                    ).wait()
                    start_gather(nxt, slot)

        for slot in range(_NBUF):
            pltpu.make_async_copy(
                rows_v[slot], out_hbm.at[pl.ds(base, _CHUNK)], wsem[slot]
            ).wait()

    return gather_kernel(table, idx_flat)


def _epilogue_body(gh_ref, gt_ref, x_ref, bh_ref, bt_ref, wh_ref, wt_ref, fb_ref, o_ref):
    n_pool = gh_ref.shape[1]
    bb = gh_ref.shape[2]
    emb = wh_ref.shape[2]
    m = (x_ref[...] != 0).astype(jnp.float32)
    bias_h = bh_ref[...]
    bias_t = bt_ref[...]
    out = jnp.zeros((bb, emb), jnp.float32)
    sq = jnp.zeros((bb,), jnp.float32)
    for p in range(n_pool):
        hs = []
        for j in range(2):
            mk = m[:, j * n_pool + p : j * n_pool + p + 1]
            sh = gh_ref[j, p] * mk
            st = gt_ref[j, p] * mk
            for g in range(1, 3):
                k = 2 * g + j
                mk = m[:, k * n_pool + p : k * n_pool + p + 1]
                sh = sh + gh_ref[k, p] * mk
                st = st + gt_ref[k, p] * mk
            hs.append(
                (jnp.maximum(sh + bias_h, 0.0), jnp.maximum(st + bias_t, 0.0))
            )
        ph = 0.5 * (hs[0][0] + hs[1][0])
        pt = 0.5 * (hs[0][1] + hs[1][1])
        sq = sq + jnp.sum(ph * ph, axis=1) + jnp.sum(pt * pt, axis=1)
        out = out + jnp.dot(ph, wh_ref[p], preferred_element_type=jnp.float32)
        out = out + jnp.dot(pt, wt_ref[p], preferred_element_type=jnp.float32)
    t = jax.lax.rsqrt(1.0 + sq)
    o_ref[...] = out * t[:, None] + fb_ref[...]


def kernel(x, embed, bias, fc_w, fc_b):
    b, sent_len, ngram = x.shape
    d, co = embed.shape
    emb = fc_w.shape[0]
    n_pool = 10
    k = sent_len // n_pool

    x = x.astype(jnp.int32)
    xg = (
        x.transpose(2, 1, 0)
        .reshape(ngram, n_pool, k, b)
        .transpose(0, 2, 1, 3)
        .reshape(ngram * k, n_pool, b)
    )
    idx_flat = xg.reshape(ngram * k * n_pool * b)
    xq = xg.reshape(ngram * k * n_pool, b).transpose(1, 0)

    tbl_a = embed[:, :_HEAD]
    tbl_b = jnp.pad(embed[:, _HEAD:], ((0, 0), (0, _TAILW - (co - _HEAD))))

    g_head = _sc_gather_ring(tbl_a, idx_flat)
    g_tail = _sc_gather_ring(tbl_b, idx_flat)
    g_head = g_head.reshape(ngram * k, n_pool, b, _HEAD)
    g_tail = g_tail.reshape(ngram * k, n_pool, b, _TAILW)

    wt = fc_w.reshape(emb, co, n_pool).transpose(2, 1, 0)
    wt_h = wt[:, :_HEAD, :]
    wt_t = jnp.pad(wt[:, _HEAD:, :], ((0, 0), (0, _TAILW - (co - _HEAD)), (0, 0)))
    bias_h = bias.reshape(1, co)[:, :_HEAD]
    bias_t = jnp.pad(bias.reshape(1, co)[:, _HEAD:], ((0, 0), (0, _TAILW - (co - _HEAD))))

    bb = 32
    out = pl.pallas_call(
        _epilogue_body,
        grid=(b // bb,),
        in_specs=[
            pl.BlockSpec((ngram * k, n_pool, bb, _HEAD), lambda i: (0, 0, i, 0)),
            pl.BlockSpec((ngram * k, n_pool, bb, _TAILW), lambda i: (0, 0, i, 0)),
            pl.BlockSpec((bb, ngram * k * n_pool), lambda i: (i, 0)),
            pl.BlockSpec((1, _HEAD), lambda i: (0, 0)),
            pl.BlockSpec((1, _TAILW), lambda i: (0, 0)),
            pl.BlockSpec((n_pool, _HEAD, emb), lambda i: (0, 0, 0)),
            pl.BlockSpec((n_pool, _TAILW, emb), lambda i: (0, 0, 0)),
            pl.BlockSpec((1, emb), lambda i: (0, 0)),
        ],
        out_specs=pl.BlockSpec((bb, emb), lambda i: (i, 0)),
        out_shape=jax.ShapeDtypeStruct((b, emb), jnp.float32),
    )(g_head, g_tail, xq, bias_h, bias_t, wt_h, wt_t, fc_b.reshape(1, emb))
    return out


# speedup vs baseline: 1.9146x; 1.9146x over previous
import jax
import jax.numpy as jnp
from jax.experimental import pallas as pl
from jax.experimental.pallas import tpu as pltpu
from jax.experimental.pallas import tpu_sc as plsc

_NC, _NS = 2, 16
_CHUNK = 32
_NBUF = 3
_HEAD = 896
_TAILW = 128


def _sc_gather2(table, tail, idx_flat):
    nidx = idx_flat.shape[0]
    nw = _NC * _NS
    per_w = nidx // nw
    rounds = per_w // _CHUNK
    mesh = plsc.VectorSubcoreMesh(core_axis_name="c", subcore_axis_name="s")

    scratch = []
    for _ in range(_NBUF):
        scratch += [
            pltpu.VMEM((_CHUNK,), jnp.int32),
            pltpu.VMEM((_CHUNK, _HEAD), jnp.float32),
            pltpu.VMEM((_CHUNK, _TAILW), jnp.float32),
            pltpu.SemaphoreType.DMA,
            pltpu.SemaphoreType.DMA,
            pltpu.SemaphoreType.DMA,
            pltpu.SemaphoreType.DMA,
        ]

    @pl.kernel(
        out_type=(
            jax.ShapeDtypeStruct((nidx, _HEAD), jnp.float32),
            jax.ShapeDtypeStruct((nidx, _TAILW), jnp.float32),
        ),
        mesh=mesh,
        scratch_types=scratch,
    )
    def gather_kernel(tbl_hbm, tail_hbm, idx_hbm, oh_hbm, ot_hbm, *bufs):
        idx_v = bufs[0::7]
        rows_h = bufs[1::7]
        rows_t = bufs[2::7]
        gsh = bufs[3::7]
        gst = bufs[4::7]
        wsh = bufs[5::7]
        wst = bufs[6::7]
        wid = jax.lax.axis_index("s") * _NC + jax.lax.axis_index("c")
        base = wid * per_w

        def start_gather(rnd, slot):
            off = base + rnd * _CHUNK
            pltpu.sync_copy(idx_hbm.at[pl.ds(off, _CHUNK)], idx_v[slot])
            pltpu.async_copy(
                tbl_hbm.at[idx_v[slot], pl.ds(0, _HEAD)], rows_h[slot], gsh[slot]
            )
            pltpu.async_copy(tail_hbm.at[idx_v[slot]], rows_t[slot], gst[slot])

        for slot in range(_NBUF):
            start_gather(slot, slot)

        @pl.loop(0, rounds // _NBUF)
        def _(t):
            cur0 = t * _NBUF
            for slot in range(_NBUF):
                cur = cur0 + slot
                off = base + cur * _CHUNK
                pltpu.make_async_copy(
                    tbl_hbm.at[idx_v[slot], pl.ds(0, _HEAD)], rows_h[slot], gsh[slot]
                ).wait()
                pltpu.make_async_copy(
                    tail_hbm.at[idx_v[slot]], rows_t[slot], gst[slot]
                ).wait()
                pltpu.async_copy(rows_h[slot], oh_hbm.at[pl.ds(off, _CHUNK)], wsh[slot])
                pltpu.async_copy(rows_t[slot], ot_hbm.at[pl.ds(off, _CHUNK)], wst[slot])
                nxt = cur + _NBUF

                @pl.when(nxt < rounds)
                def _():
                    pltpu.make_async_copy(
                        rows_h[slot], oh_hbm.at[pl.ds(base, _CHUNK)], wsh[slot]
                    ).wait()
                    pltpu.make_async_copy(
                        rows_t[slot], ot_hbm.at[pl.ds(base, _CHUNK)], wst[slot]
                    ).wait()
                    start_gather(nxt, slot)

        for slot in range(_NBUF):
            pltpu.make_async_copy(
                rows_h[slot], oh_hbm.at[pl.ds(base, _CHUNK)], wsh[slot]
            ).wait()
            pltpu.make_async_copy(
                rows_t[slot], ot_hbm.at[pl.ds(base, _CHUNK)], wst[slot]
            ).wait()

    return gather_kernel(table, tail, idx_flat)


def _epilogue_body(gh_ref, gt_ref, x_ref, bh_ref, bt_ref, wh_ref, wt_ref, fb_ref, o_ref):
    n_pool = gh_ref.shape[1]
    bb = gh_ref.shape[2]
    emb = wh_ref.shape[2]
    m = (x_ref[...] != 0).astype(jnp.float32)
    bias_h = bh_ref[...]
    bias_t = bt_ref[...]
    out = jnp.zeros((bb, emb), jnp.float32)
    sq = jnp.zeros((bb,), jnp.float32)
    for p in range(n_pool):
        hs = []
        for j in range(2):
            mk = m[:, j * n_pool + p : j * n_pool + p + 1]
            sh = gh_ref[j, p] * mk
            st = gt_ref[j, p] * mk
            for g in range(1, 3):
                k = 2 * g + j
                mk = m[:, k * n_pool + p : k * n_pool + p + 1]
                sh = sh + gh_ref[k, p] * mk
                st = st + gt_ref[k, p] * mk
            hs.append(
                (jnp.maximum(sh + bias_h, 0.0), jnp.maximum(st + bias_t, 0.0))
            )
        ph = 0.5 * (hs[0][0] + hs[1][0])
        pt = 0.5 * (hs[0][1] + hs[1][1])
        sq = sq + jnp.sum(ph * ph, axis=1) + jnp.sum(pt * pt, axis=1)
        out = out + jnp.dot(ph, wh_ref[p], preferred_element_type=jnp.float32)
        out = out + jnp.dot(pt, wt_ref[p], preferred_element_type=jnp.float32)
    t = jax.lax.rsqrt(1.0 + sq)
    o_ref[...] = out * t[:, None] + fb_ref[...]


def kernel(x, embed, bias, fc_w, fc_b):
    b, sent_len, ngram = x.shape
    d, co = embed.shape
    emb = fc_w.shape[0]
    n_pool = 10
    k = sent_len // n_pool

    x = x.astype(jnp.int32)
    xg = (
        x.transpose(2, 1, 0)
        .reshape(ngram, n_pool, k, b)
        .transpose(0, 2, 1, 3)
        .reshape(ngram * k, n_pool, b)
    )
    idx_flat = xg.reshape(ngram * k * n_pool * b)
    xq = xg.reshape(ngram * k * n_pool, b).transpose(1, 0)

    tail = jnp.pad(embed[:, _HEAD:], ((0, 0), (0, _TAILW - (co - _HEAD))))

    g_head, g_tail = _sc_gather2(embed, tail, idx_flat)
    g_head = g_head.reshape(ngram * k, n_pool, b, _HEAD)
    g_tail = g_tail.reshape(ngram * k, n_pool, b, _TAILW)

    wt = fc_w.reshape(emb, co, n_pool).transpose(2, 1, 0)
    wt_h = wt[:, :_HEAD, :]
    wt_t = jnp.pad(wt[:, _HEAD:, :], ((0, 0), (0, _TAILW - (co - _HEAD)), (0, 0)))
    bias_h = bias.reshape(1, co)[:, :_HEAD]
    bias_t = jnp.pad(bias.reshape(1, co)[:, _HEAD:], ((0, 0), (0, _TAILW - (co - _HEAD))))

    bb = 32
    out = pl.pallas_call(
        _epilogue_body,
        grid=(b // bb,),
        in_specs=[
            pl.BlockSpec((ngram * k, n_pool, bb, _HEAD), lambda i: (0, 0, i, 0)),
            pl.BlockSpec((ngram * k, n_pool, bb, _TAILW), lambda i: (0, 0, i, 0)),
            pl.BlockSpec((bb, ngram * k * n_pool), lambda i: (i, 0)),
            pl.BlockSpec((1, _HEAD), lambda i: (0, 0)),
            pl.BlockSpec((1, _TAILW), lambda i: (0, 0)),
            pl.BlockSpec((n_pool, _HEAD, emb), lambda i: (0, 0, 0)),
            pl.BlockSpec((n_pool, _TAILW, emb), lambda i: (0, 0, 0)),
            pl.BlockSpec((1, emb), lambda i: (0, 0)),
        ],
        out_specs=pl.BlockSpec((bb, emb), lambda i: (i, 0)),
        out_shape=jax.ShapeDtypeStruct((b, emb), jnp.float32),
    )(g_head, g_tail, xq, bias_h, bias_t, wt_h, wt_t, fc_b.reshape(1, emb))
    return out


# speedup vs baseline: 1.9174x; 1.0015x over previous
import jax
import jax.numpy as jnp
from jax.experimental import pallas as pl
from jax.experimental.pallas import tpu as pltpu
from jax.experimental.pallas import tpu_sc as plsc

_NC, _NS = 2, 16
_CHUNK = 48
_NBUF = 2
_HEAD = 896
_TAILW = 128


def _sc_gather2(table, tail, idx_flat):
    nidx = idx_flat.shape[0]
    nw = _NC * _NS
    per_w = nidx // nw
    rounds = per_w // _CHUNK
    mesh = plsc.VectorSubcoreMesh(core_axis_name="c", subcore_axis_name="s")

    scratch = []
    for _ in range(_NBUF):
        scratch += [
            pltpu.VMEM((_CHUNK,), jnp.int32),
            pltpu.VMEM((_CHUNK, _HEAD), jnp.float32),
            pltpu.VMEM((_CHUNK, _TAILW), jnp.float32),
            pltpu.SemaphoreType.DMA,
            pltpu.SemaphoreType.DMA,
            pltpu.SemaphoreType.DMA,
            pltpu.SemaphoreType.DMA,
        ]

    @pl.kernel(
        out_type=(
            jax.ShapeDtypeStruct((nidx, _HEAD), jnp.float32),
            jax.ShapeDtypeStruct((nidx, _TAILW), jnp.float32),
        ),
        mesh=mesh,
        scratch_types=scratch,
    )
    def gather_kernel(tbl_hbm, tail_hbm, idx_hbm, oh_hbm, ot_hbm, *bufs):
        idx_v = bufs[0::7]
        rows_h = bufs[1::7]
        rows_t = bufs[2::7]
        gsh = bufs[3::7]
        gst = bufs[4::7]
        wsh = bufs[5::7]
        wst = bufs[6::7]
        wid = jax.lax.axis_index("s") * _NC + jax.lax.axis_index("c")
        base = wid * per_w

        def start_gather(rnd, slot):
            off = base + rnd * _CHUNK
            pltpu.sync_copy(idx_hbm.at[pl.ds(off, _CHUNK)], idx_v[slot])
            pltpu.async_copy(
                tbl_hbm.at[idx_v[slot], pl.ds(0, _HEAD)], rows_h[slot], gsh[slot]
            )
            pltpu.async_copy(tail_hbm.at[idx_v[slot]], rows_t[slot], gst[slot])

        for slot in range(_NBUF):
            start_gather(slot, slot)

        @pl.loop(0, rounds // _NBUF)
        def _(t):
            cur0 = t * _NBUF
            for slot in range(_NBUF):
                cur = cur0 + slot
                off = base + cur * _CHUNK
                pltpu.make_async_copy(
                    tbl_hbm.at[idx_v[slot], pl.ds(0, _HEAD)], rows_h[slot], gsh[slot]
                ).wait()
                pltpu.make_async_copy(
                    tail_hbm.at[idx_v[slot]], rows_t[slot], gst[slot]
                ).wait()
                pltpu.async_copy(rows_h[slot], oh_hbm.at[pl.ds(off, _CHUNK)], wsh[slot])
                pltpu.async_copy(rows_t[slot], ot_hbm.at[pl.ds(off, _CHUNK)], wst[slot])
                nxt = cur + _NBUF

                @pl.when(nxt < rounds)
                def _():
                    pltpu.make_async_copy(
                        rows_h[slot], oh_hbm.at[pl.ds(base, _CHUNK)], wsh[slot]
                    ).wait()
                    pltpu.make_async_copy(
                        rows_t[slot], ot_hbm.at[pl.ds(base, _CHUNK)], wst[slot]
                    ).wait()
                    start_gather(nxt, slot)

        for slot in range(_NBUF):
            pltpu.make_async_copy(
                rows_h[slot], oh_hbm.at[pl.ds(base, _CHUNK)], wsh[slot]
            ).wait()
            pltpu.make_async_copy(
                rows_t[slot], ot_hbm.at[pl.ds(base, _CHUNK)], wst[slot]
            ).wait()

    return gather_kernel(table, tail, idx_flat)


def _epilogue_body(gh_ref, gt_ref, x_ref, bh_ref, bt_ref, wh_ref, wt_ref, fb_ref, o_ref):
    n_pool = gh_ref.shape[1]
    bb = gh_ref.shape[2]
    emb = wh_ref.shape[2]
    m = (x_ref[...] != 0).astype(jnp.float32)
    bias_h = bh_ref[...]
    bias_t = bt_ref[...]
    out = jnp.zeros((bb, emb), jnp.float32)
    sq = jnp.zeros((bb,), jnp.float32)
    for p in range(n_pool):
        hs = []
        for j in range(2):
            mk = m[:, j * n_pool + p : j * n_pool + p + 1]
            sh = gh_ref[j, p] * mk
            st = gt_ref[j, p] * mk
            for g in range(1, 3):
                k = 2 * g + j
                mk = m[:, k * n_pool + p : k * n_pool + p + 1]
                sh = sh + gh_ref[k, p] * mk
                st = st + gt_ref[k, p] * mk
            hs.append(
                (jnp.maximum(sh + bias_h, 0.0), jnp.maximum(st + bias_t, 0.0))
            )
        ph = 0.5 * (hs[0][0] + hs[1][0])
        pt = 0.5 * (hs[0][1] + hs[1][1])
        sq = sq + jnp.sum(ph * ph, axis=1) + jnp.sum(pt * pt, axis=1)
        out = out + jnp.dot(ph, wh_ref[p], preferred_element_type=jnp.float32)
        out = out + jnp.dot(pt, wt_ref[p], preferred_element_type=jnp.float32)
    t = jax.lax.rsqrt(1.0 + sq)
    o_ref[...] = out * t[:, None] + fb_ref[...]


def kernel(x, embed, bias, fc_w, fc_b):
    b, sent_len, ngram = x.shape
    d, co = embed.shape
    emb = fc_w.shape[0]
    n_pool = 10
    k = sent_len // n_pool

    x = x.astype(jnp.int32)
    xg = (
        x.transpose(2, 1, 0)
        .reshape(ngram, n_pool, k, b)
        .transpose(0, 2, 1, 3)
        .reshape(ngram * k, n_pool, b)
    )
    idx_flat = xg.reshape(ngram * k * n_pool * b)
    xq = xg.reshape(ngram * k * n_pool, b).transpose(1, 0)

    tail = jnp.pad(embed[:, _HEAD:], ((0, 0), (0, _TAILW - (co - _HEAD))))

    g_head, g_tail = _sc_gather2(embed, tail, idx_flat)
    g_head = g_head.reshape(ngram * k, n_pool, b, _HEAD)
    g_tail = g_tail.reshape(ngram * k, n_pool, b, _TAILW)

    wt = fc_w.reshape(emb, co, n_pool).transpose(2, 1, 0)
    wt_h = wt[:, :_HEAD, :]
    wt_t = jnp.pad(wt[:, _HEAD:, :], ((0, 0), (0, _TAILW - (co - _HEAD)), (0, 0)))
    bias_h = bias.reshape(1, co)[:, :_HEAD]
    bias_t = jnp.pad(bias.reshape(1, co)[:, _HEAD:], ((0, 0), (0, _TAILW - (co - _HEAD))))

    bb = 32
    out = pl.pallas_call(
        _epilogue_body,
        grid=(b // bb,),
        in_specs=[
            pl.BlockSpec((ngram * k, n_pool, bb, _HEAD), lambda i: (0, 0, i, 0)),
            pl.BlockSpec((ngram * k, n_pool, bb, _TAILW), lambda i: (0, 0, i, 0)),
            pl.BlockSpec((bb, ngram * k * n_pool), lambda i: (i, 0)),
            pl.BlockSpec((1, _HEAD), lambda i: (0, 0)),
            pl.BlockSpec((1, _TAILW), lambda i: (0, 0)),
            pl.BlockSpec((n_pool, _HEAD, emb), lambda i: (0, 0, 0)),
            pl.BlockSpec((n_pool, _TAILW, emb), lambda i: (0, 0, 0)),
            pl.BlockSpec((1, emb), lambda i: (0, 0)),
        ],
        out_specs=pl.BlockSpec((bb, emb), lambda i: (i, 0)),
        out_shape=jax.ShapeDtypeStruct((b, emb), jnp.float32),
    )(g_head, g_tail, xq, bias_h, bias_t, wt_h, wt_t, fc_b.reshape(1, emb))
    return out


# speedup vs baseline: 1.9319x; 1.0076x over previous
import jax
import jax.numpy as jnp
from jax.experimental import pallas as pl
from jax.experimental.pallas import tpu as pltpu
from jax.experimental.pallas import tpu_sc as plsc

_NC, _NS = 2, 16
_CHUNK = 48
_NBUF = 2
_HEAD = 896
_TAILW = 128


def _sc_gather2(table, tail, idx_flat):
    nidx = idx_flat.shape[0]
    nw = _NC * _NS
    per_w = nidx // nw
    rounds = per_w // _CHUNK
    mesh = plsc.VectorSubcoreMesh(core_axis_name="c", subcore_axis_name="s")

    scratch = []
    for _ in range(_NBUF):
        scratch += [
            pltpu.VMEM((_CHUNK,), jnp.int32),
            pltpu.VMEM((_CHUNK, _HEAD), jnp.float32),
            pltpu.VMEM((_CHUNK, _TAILW), jnp.float32),
            pltpu.SemaphoreType.DMA,
            pltpu.SemaphoreType.DMA,
            pltpu.SemaphoreType.DMA,
            pltpu.SemaphoreType.DMA,
        ]

    @pl.kernel(
        out_type=(
            jax.ShapeDtypeStruct((nidx, _HEAD), jnp.float32),
            jax.ShapeDtypeStruct((nidx, _TAILW), jnp.float32),
        ),
        mesh=mesh,
        scratch_types=scratch,
    )
    def gather_kernel(tbl_hbm, tail_hbm, idx_hbm, oh_hbm, ot_hbm, *bufs):
        idx_v = bufs[0::7]
        rows_h = bufs[1::7]
        rows_t = bufs[2::7]
        gsh = bufs[3::7]
        gst = bufs[4::7]
        wsh = bufs[5::7]
        wst = bufs[6::7]
        wid = jax.lax.axis_index("s") * _NC + jax.lax.axis_index("c")
        base = wid * per_w

        def start_gather(rnd, slot):
            off = base + rnd * _CHUNK
            pltpu.sync_copy(idx_hbm.at[pl.ds(off, _CHUNK)], idx_v[slot])
            pltpu.async_copy(
                tbl_hbm.at[idx_v[slot], pl.ds(0, _HEAD)], rows_h[slot], gsh[slot]
            )
            pltpu.async_copy(tail_hbm.at[idx_v[slot]], rows_t[slot], gst[slot])

        for slot in range(_NBUF):
            start_gather(slot, slot)

        @pl.loop(0, rounds // _NBUF)
        def _(t):
            cur0 = t * _NBUF
            for slot in range(_NBUF):
                cur = cur0 + slot
                off = base + cur * _CHUNK
                pltpu.make_async_copy(
                    tbl_hbm.at[idx_v[slot], pl.ds(0, _HEAD)], rows_h[slot], gsh[slot]
                ).wait()
                pltpu.make_async_copy(
                    tail_hbm.at[idx_v[slot]], rows_t[slot], gst[slot]
                ).wait()
                pltpu.async_copy(rows_h[slot], oh_hbm.at[pl.ds(off, _CHUNK)], wsh[slot])
                pltpu.async_copy(rows_t[slot], ot_hbm.at[pl.ds(off, _CHUNK)], wst[slot])
                nxt = cur + _NBUF

                @pl.when(nxt < rounds)
                def _():
                    pltpu.make_async_copy(
                        rows_h[slot], oh_hbm.at[pl.ds(base, _CHUNK)], wsh[slot]
                    ).wait()
                    pltpu.make_async_copy(
                        rows_t[slot], ot_hbm.at[pl.ds(base, _CHUNK)], wst[slot]
                    ).wait()
                    start_gather(nxt, slot)

        for slot in range(_NBUF):
            pltpu.make_async_copy(
                rows_h[slot], oh_hbm.at[pl.ds(base, _CHUNK)], wsh[slot]
            ).wait()
            pltpu.make_async_copy(
                rows_t[slot], ot_hbm.at[pl.ds(base, _CHUNK)], wst[slot]
            ).wait()

    return gather_kernel(table, tail, idx_flat)


def _epilogue_body(gh_ref, gt_ref, x_ref, bh_ref, bt_ref, wh_ref, wt_ref, fb_ref, o_ref):
    n_pool = gh_ref.shape[1]
    bb = gh_ref.shape[2]
    emb = wh_ref.shape[2]
    m = (x_ref[...] != 0).astype(jnp.float32)
    bias_h = bh_ref[...]
    bias_t = bt_ref[...]
    out = jnp.zeros((bb, emb), jnp.float32)
    sq = jnp.zeros((bb,), jnp.float32)
    for p in range(n_pool):
        hs = []
        for j in range(2):
            mk = m[:, j * n_pool + p : j * n_pool + p + 1]
            sh = gh_ref[j, p] * mk
            st = gt_ref[j, p] * mk
            for g in range(1, 3):
                k = 2 * g + j
                mk = m[:, k * n_pool + p : k * n_pool + p + 1]
                sh = sh + gh_ref[k, p] * mk
                st = st + gt_ref[k, p] * mk
            hs.append(
                (jnp.maximum(sh + bias_h, 0.0), jnp.maximum(st + bias_t, 0.0))
            )
        ph = 0.5 * (hs[0][0] + hs[1][0])
        pt = 0.5 * (hs[0][1] + hs[1][1])
        sq = sq + jnp.sum(ph * ph, axis=1) + jnp.sum(pt * pt, axis=1)
        out = out + jnp.dot(ph, wh_ref[p], preferred_element_type=jnp.float32)
        out = out + jnp.dot(pt, wt_ref[p], preferred_element_type=jnp.float32)
    t = jax.lax.rsqrt(1.0 + sq)
    o_ref[...] = out * t[:, None] + fb_ref[...]


def kernel(x, embed, bias, fc_w, fc_b):
    b, sent_len, ngram = x.shape
    d, co = embed.shape
    emb = fc_w.shape[0]
    n_pool = 10
    k = sent_len // n_pool

    x = x.astype(jnp.int32)
    xg = (
        x.transpose(2, 1, 0)
        .reshape(ngram, n_pool, k, b)
        .transpose(0, 2, 1, 3)
        .reshape(ngram * k, n_pool, b)
    )
    idx_flat = xg.reshape(ngram * k * n_pool * b)
    xq = xg.reshape(ngram * k * n_pool, b).transpose(1, 0)

    tail = jnp.pad(embed[:, _HEAD:], ((0, 0), (0, _TAILW - (co - _HEAD))))

    g_head, g_tail = _sc_gather2(embed, tail, idx_flat)
    g_head = g_head.reshape(ngram * k, n_pool, b, _HEAD)
    g_tail = g_tail.reshape(ngram * k, n_pool, b, _TAILW)

    wt = fc_w.reshape(emb, co, n_pool).transpose(2, 1, 0)
    wt_h = wt[:, :_HEAD, :]
    wt_t = jnp.pad(wt[:, _HEAD:, :], ((0, 0), (0, _TAILW - (co - _HEAD)), (0, 0)))
    bias_h = bias.reshape(1, co)[:, :_HEAD]
    bias_t = jnp.pad(bias.reshape(1, co)[:, _HEAD:], ((0, 0), (0, _TAILW - (co - _HEAD))))

    bb = 64
    out = pl.pallas_call(
        _epilogue_body,
        grid=(b // bb,),
        in_specs=[
            pl.BlockSpec((ngram * k, n_pool, bb, _HEAD), lambda i: (0, 0, i, 0)),
            pl.BlockSpec((ngram * k, n_pool, bb, _TAILW), lambda i: (0, 0, i, 0)),
            pl.BlockSpec((bb, ngram * k * n_pool), lambda i: (i, 0)),
            pl.BlockSpec((1, _HEAD), lambda i: (0, 0)),
            pl.BlockSpec((1, _TAILW), lambda i: (0, 0)),
            pl.BlockSpec((n_pool, _HEAD, emb), lambda i: (0, 0, 0)),
            pl.BlockSpec((n_pool, _TAILW, emb), lambda i: (0, 0, 0)),
            pl.BlockSpec((1, emb), lambda i: (0, 0)),
        ],
        out_specs=pl.BlockSpec((bb, emb), lambda i: (i, 0)),
        out_shape=jax.ShapeDtypeStruct((b, emb), jnp.float32),
    )(g_head, g_tail, xq, bias_h, bias_t, wt_h, wt_t, fc_b.reshape(1, emb))
    return out
